# baseline (speedup 1.0000x reference)
import functools

import jax
import jax.numpy as jnp
from jax import lax
from jax.experimental import pallas as pl
from jax.experimental.pallas import tpu as pltpu
from jax.experimental.pallas import tpu_sc as plsc

N = 10000
F_IN = 128
H = 32
DZ = 16

NC, NS = 2, 16
NW = NC * NS
CHUNK = 128
K_PIPE = 6
ACC_ROWS = 10112
RPS = ACC_ROWS // NS
PR = ACC_ROWS // 4


def _deg_body(edge_hbm, ones_hbm, zeros_hbm, out_hbm, dst_v, dst_t, ones_v,
              acc, isem, ssem):
    cid = lax.axis_index("c")
    sid = lax.axis_index("s")
    wid = sid * NC + cid
    epw = edge_hbm.shape[1] // NW
    nf = epw // CHUNK
    tail = epw - nf * CHUNK
    groups = nf // K_PIPE
    r0 = sid * RPS
    ew0 = wid * epw
    pltpu.sync_copy(zeros_hbm.at[pl.ds(r0, RPS)], acc.at[pl.ds(r0, RPS)])
    pltpu.sync_copy(ones_hbm, ones_v)
    plsc.subcore_barrier()

    def body(g, carry):
        base0 = ew0 + g * (K_PIPE * CHUNK)
        ih = [pltpu.async_copy(edge_hbm.at[1, pl.ds(base0 + b * CHUNK, CHUNK)],
                               dst_v.at[b], isem) for b in range(K_PIPE)]
        for h in ih:
            h.wait()
        sh = [pltpu.async_copy(ones_v, acc.at[dst_v.at[b]], ssem, add=True)
              for b in range(K_PIPE)]
        for h in sh:
            h.wait()
        return carry

    lax.fori_loop(0, groups, body, 0)
    if tail:
        pltpu.sync_copy(edge_hbm.at[1, pl.ds(ew0 + nf * CHUNK, tail)], dst_t)
        pltpu.sync_copy(ones_v.at[pl.ds(0, tail)], acc.at[dst_t], add=True)
    plsc.subcore_barrier()
    pltpu.sync_copy(acc.at[pl.ds(r0, RPS)],
                    out_hbm.at[pl.ds(cid * ACC_ROWS + r0, RPS)])


def _segsum_body(table_hbm, edge_hbm, zeros_hbm, out_hbm,
                 src_v, dst_v, rows_v, src_t, dst_t, rows_t, acc, tbl,
                 isem, gsem, ssem):
    cid = lax.axis_index("c")
    sid = lax.axis_index("s")
    wid = sid * NC + cid
    n_tbl = table_hbm.shape[0]
    tps = n_tbl // NS
    epw = edge_hbm.shape[1] // NW
    nf = epw // CHUNK
    tail = epw - nf * CHUNK
    groups = nf // K_PIPE
    r0 = sid * RPS
    ew0 = wid * epw
    pltpu.sync_copy(zeros_hbm.at[pl.ds(r0, RPS)], acc.at[pl.ds(r0, RPS)])
    t0 = sid * tps
    pltpu.sync_copy(table_hbm.at[pl.ds(t0, tps)], tbl.at[pl.ds(t0, tps)])
    plsc.subcore_barrier()

    def body(g, carry):
        base0 = ew0 + g * (K_PIPE * CHUNK)
        ih = []
        for b in range(K_PIPE):
            ih.append(pltpu.async_copy(
                edge_hbm.at[0, pl.ds(base0 + b * CHUNK, CHUNK)],
                src_v.at[b], isem))
            ih.append(pltpu.async_copy(
                edge_hbm.at[1, pl.ds(base0 + b * CHUNK, CHUNK)],
                dst_v.at[b], isem))
        for h in ih:
            h.wait()
        gh = [pltpu.async_copy(tbl.at[src_v.at[b]], rows_v.at[b], gsem)
              for b in range(K_PIPE)]
        for h in gh:
            h.wait()
        sh = [pltpu.async_copy(rows_v.at[b], acc.at[dst_v.at[b]], ssem,
                               add=True) for b in range(K_PIPE)]
        for h in sh:
            h.wait()
        return carry

    lax.fori_loop(0, groups, body, 0)
    if tail:
        base = ew0 + nf * CHUNK
        pltpu.sync_copy(edge_hbm.at[0, pl.ds(base, tail)], src_t)
        pltpu.sync_copy(edge_hbm.at[1, pl.ds(base, tail)], dst_t)
        pltpu.async_copy(tbl.at[src_t], rows_t, gsem).wait()
        pltpu.sync_copy(rows_t, acc.at[dst_t], add=True)
    plsc.subcore_barrier()
    pltpu.sync_copy(acc.at[pl.ds(r0, RPS)],
                    out_hbm.at[pl.ds(cid * ACC_ROWS + r0, RPS)])


@functools.cache
def _sc_kernels(e, n_tbl):
    epw = e // NW
    tail = max(epw - (epw // CHUNK) * CHUNK, 8)
    mesh = plsc.VectorSubcoreMesh(
        core_axis_name="c", subcore_axis_name="s",
        num_cores=NC, num_subcores=NS)
    params = pltpu.CompilerParams(use_tc_tiling_on_sc=False)
    deg_kernel = pl.kernel(
        _deg_body,
        out_type=jax.ShapeDtypeStruct((NC * ACC_ROWS, H), jnp.float32),
        mesh=mesh,
        compiler_params=params,
        scratch_types=[
            pltpu.VMEM((K_PIPE, CHUNK), jnp.int32),
            pltpu.VMEM((tail,), jnp.int32),
            pltpu.VMEM((CHUNK, H), jnp.float32),
            pltpu.VMEM_SHARED((ACC_ROWS, H), jnp.float32),
            pltpu.SemaphoreType.DMA,
            pltpu.SemaphoreType.DMA,
        ],
    )
    segsum_kernel = pl.kernel(
        _segsum_body,
        out_type=jax.ShapeDtypeStruct((NC * ACC_ROWS, H), jnp.float32),
        mesh=mesh,
        compiler_params=params,
        scratch_types=[
            pltpu.VMEM((K_PIPE, CHUNK), jnp.int32),
            pltpu.VMEM((K_PIPE, CHUNK), jnp.int32),
            pltpu.VMEM((K_PIPE, CHUNK, H), jnp.float32),
            pltpu.VMEM((tail,), jnp.int32),
            pltpu.VMEM((tail,), jnp.int32),
            pltpu.VMEM((tail, H), jnp.float32),
            pltpu.VMEM_SHARED((ACC_ROWS, H), jnp.float32),
            pltpu.VMEM_SHARED((n_tbl, H), jnp.float32),
            pltpu.SemaphoreType.DMA,
            pltpu.SemaphoreType.DMA,
            pltpu.SemaphoreType.DMA,
        ],
    )
    return deg_kernel, segsum_kernel


BLKP = PR // 4


def _norm_phys(d0, d1):
    deg = d0 + d1
    return jnp.where(deg > 0.0, lax.rsqrt(deg), 0.0)


def _k1_body(d0_ref, d1_ref, x4_ref, w4_ref, a1_ref):
    norm = _norm_phys(d0_ref[...], d1_ref[...])
    a1_ref[...] = jnp.dot(x4_ref[...], w4_ref[...],
                          preferred_element_type=jnp.float32) * norm


def _k2_body(u0_ref, u1_ref, d0_ref, d1_ref, w4_ref, a2_ref):
    deg = d0_ref[...] + d1_ref[...]
    inv = jnp.where(deg > 0.0, 1.0 / deg, 0.0)
    u = u0_ref[...] + u1_ref[...]
    a2_ref[...] = jnp.dot(u, w4_ref[...],
                          preferred_element_type=jnp.float32) * inv


def _k3_body(u0_ref, u1_ref, d0_ref, d1_ref, n_ref, y_ref):
    norm = _norm_phys(d0_ref[...], d1_ref[...])
    g = jnp.maximum((u0_ref[...] + u1_ref[...]) * norm, 0.0)
    gs = jnp.concatenate([g[:, 16:], g[:, :16]], axis=1)
    lane = lax.broadcasted_iota(jnp.int32, g.shape, 1)
    y_ref[...] = jnp.where((lane % 32) < 16,
                           n_ref[...] * jnp.exp(gs) + g, 0.0)


BM = 512


def _k4_body(ytb_ref, yt_ref, out_ref):
    out_ref[...] = lax.dot_general(ytb_ref[...], yt_ref[...],
                                   (((0,), (0,)), ((), ())),
                                   preferred_element_type=jnp.float32)


def _dense_stage1(d0, d1, x4, w4):
    return pl.pallas_call(
        _k1_body,
        grid=(PR // BLKP,),
        in_specs=[
            pl.BlockSpec((BLKP, 128), lambda i: (i, 0)),
            pl.BlockSpec((BLKP, 128), lambda i: (i, 0)),
            pl.BlockSpec((BLKP, 4 * F_IN), lambda i: (i, 0)),
            pl.BlockSpec((4 * F_IN, 128), lambda i: (0, 0)),
        ],
        out_specs=pl.BlockSpec((BLKP, 128), lambda i: (i, 0)),
        out_shape=jax.ShapeDtypeStruct((PR, 128), jnp.float32),
    )(d0, d1, x4, w4)


def _dense_stage2(u0, u1, d0, d1, w4):
    return pl.pallas_call(
        _k2_body,
        grid=(PR // BLKP,),
        in_specs=[
            pl.BlockSpec((BLKP, 128), lambda i: (i, 0)),
            pl.BlockSpec((BLKP, 128), lambda i: (i, 0)),
            pl.BlockSpec((BLKP, 128), lambda i: (i, 0)),
            pl.BlockSpec((BLKP, 128), lambda i: (i, 0)),
            pl.BlockSpec((128, 128), lambda i: (0, 0)),
        ],
        out_specs=pl.BlockSpec((BLKP, 128), lambda i: (i, 0)),
        out_shape=jax.ShapeDtypeStruct((PR, 128), jnp.float32),
    )(u0, u1, d0, d1, w4)


def _dense_stage3(u0, u1, d0, d1, noise_p):
    return pl.pallas_call(
        _k3_body,
        grid=(PR // BLKP,),
        in_specs=[
            pl.BlockSpec((BLKP, 128), lambda i: (i, 0)),
            pl.BlockSpec((BLKP, 128), lambda i: (i, 0)),
            pl.BlockSpec((BLKP, 128), lambda i: (i, 0)),
            pl.BlockSpec((BLKP, 128), lambda i: (i, 0)),
            pl.BlockSpec((BLKP, 128), lambda i: (i, 0)),
        ],
        out_specs=pl.BlockSpec((BLKP, 128), lambda i: (i, 0)),
        out_shape=jax.ShapeDtypeStruct((PR, 128), jnp.float32),
    )(u0, u1, d0, d1, noise_p)


def _dense_stage4(yt):
    return pl.pallas_call(
        _k4_body,
        grid=(pl.cdiv(N, BM),),
        in_specs=[
            pl.BlockSpec((H, BM), lambda i: (0, i)),
            pl.BlockSpec((H, N), lambda i: (0, 0)),
        ],
        out_specs=pl.BlockSpec((BM, N), lambda i: (i, 0)),
        out_shape=jax.ShapeDtypeStruct((N, N), jnp.float32),
    )(yt, yt)


def kernel(features, edge_index, W_base, W_mean, W_logstd):
    e = edge_index.shape[1]

    zeros_h = jnp.zeros((ACC_ROWS, H), jnp.float32)
    ones_d = jnp.ones((CHUNK, H), jnp.float32)
    eye4 = jnp.eye(4, dtype=jnp.float32)
    w4b = jnp.kron(eye4, W_base)
    w4c = jnp.kron(eye4, jnp.concatenate([W_mean, W_logstd], axis=1))
    noise = jax.random.normal(jax.random.key(42), (N, DZ), jnp.float32)
    noise_p = jnp.zeros((ACC_ROWS, H), jnp.float32)
    noise_p = noise_p.at[:N, :DZ].set(noise).reshape(PR, 128)
    x4 = features.reshape(N // 4, 4 * F_IN)

    deg_kernel, segsum_kernel = _sc_kernels(e, ACC_ROWS)
    degp = deg_kernel(edge_index, ones_d, zeros_h)
    d0 = degp[:ACC_ROWS].reshape(PR, 128)
    d1 = degp[ACC_ROWS:].reshape(PR, 128)
    d0, d1 = lax.optimization_barrier((d0, d1))
    x4p = jnp.zeros((PR, 4 * F_IN), jnp.float32).at[:N // 4].set(x4)
    a1 = _dense_stage1(d0, d1, x4p, w4b).reshape(ACC_ROWS, H)
    u1p = segsum_kernel(a1, edge_index, zeros_h)
    a2 = _dense_stage2(u1p[:ACC_ROWS].reshape(PR, 128),
                       u1p[ACC_ROWS:].reshape(PR, 128),
                       d0, d1, w4c).reshape(ACC_ROWS, H)
    u2p = segsum_kernel(a2, edge_index, zeros_h)
    y = _dense_stage3(u2p[:ACC_ROWS].reshape(PR, 128),
                      u2p[ACC_ROWS:].reshape(PR, 128),
                      d0, d1, noise_p)
    yt = y.reshape(ACC_ROWS, H)[:N].T
    return _dense_stage4(yt)


# speedup vs baseline: 1.0754x; 1.0754x over previous
import functools

import jax
import jax.numpy as jnp
from jax import lax
from jax.experimental import pallas as pl
from jax.experimental.pallas import tpu as pltpu
from jax.experimental.pallas import tpu_sc as plsc

N = 10000
F_IN = 128
H = 32
DZ = 16

NC, NS = 2, 16
NW = NC * NS
CHUNK = 128
K_PIPE = 13
ACC_ROWS = 10112
RPS = ACC_ROWS // NS
PR = ACC_ROWS // 4


def _deg_body(edge_hbm, ones_hbm, zeros_hbm, out_hbm, dst_all, ones_v,
              acc, isem, ssem):
    cid = lax.axis_index("c")
    sid = lax.axis_index("s")
    wid = sid * NC + cid
    er = edge_hbm.shape[1]
    rw = er // NW
    rem = er - rw * NW
    groups = rw // K_PIPE
    r0 = sid * RPS
    w0 = wid * rw
    zh = pltpu.async_copy(zeros_hbm.at[pl.ds(r0, RPS)],
                          acc.at[pl.ds(r0, RPS)], isem)
    oh = pltpu.async_copy(ones_hbm, ones_v, isem)
    dh = pltpu.async_copy(edge_hbm.at[1, pl.ds(w0, rw)],
                          dst_all.at[pl.ds(0, rw)], isem)
    zh.wait()
    oh.wait()
    dh.wait()

    @pl.when(wid < rem)
    def _():
        pltpu.sync_copy(edge_hbm.at[1, pl.ds(NW * rw + wid, 1)],
                        dst_all.at[pl.ds(rw, 1)])

    plsc.subcore_barrier()

    def body(g, carry):
        sh = [pltpu.async_copy(ones_v, acc.at[dst_all.at[g * K_PIPE + b]],
                               ssem, add=True) for b in range(K_PIPE)]
        for h in sh:
            h.wait()
        return carry

    lax.fori_loop(0, groups, body, 0)

    @pl.when(wid < rem)
    def _():
        pltpu.sync_copy(ones_v, acc.at[dst_all.at[rw]], add=True)

    plsc.subcore_barrier()
    pltpu.sync_copy(acc.at[pl.ds(r0, RPS)],
                    out_hbm.at[pl.ds(cid * ACC_ROWS + r0, RPS)])


def _segsum_body(table_hbm, edge_hbm, zeros_hbm, out_hbm,
                 src_all, dst_all, rows_v, acc, tbl, isem, gsem, ssem):
    cid = lax.axis_index("c")
    sid = lax.axis_index("s")
    wid = sid * NC + cid
    n_tbl = table_hbm.shape[0]
    tps = n_tbl // NS
    er = edge_hbm.shape[1]
    rw = er // NW
    rem = er - rw * NW
    groups = rw // K_PIPE
    r0 = sid * RPS
    w0 = wid * rw
    zh = pltpu.async_copy(zeros_hbm.at[pl.ds(r0, RPS)],
                          acc.at[pl.ds(r0, RPS)], isem)
    t0 = sid * tps
    th = pltpu.async_copy(table_hbm.at[pl.ds(t0, tps)],
                          tbl.at[pl.ds(t0, tps)], isem)
    sh0 = pltpu.async_copy(edge_hbm.at[0, pl.ds(w0, rw)],
                           src_all.at[pl.ds(0, rw)], isem)
    dh0 = pltpu.async_copy(edge_hbm.at[1, pl.ds(w0, rw)],
                           dst_all.at[pl.ds(0, rw)], isem)
    zh.wait()
    th.wait()
    sh0.wait()
    dh0.wait()

    @pl.when(wid < rem)
    def _():
        pltpu.sync_copy(edge_hbm.at[0, pl.ds(NW * rw + wid, 1)],
                        src_all.at[pl.ds(rw, 1)])
        pltpu.sync_copy(edge_hbm.at[1, pl.ds(NW * rw + wid, 1)],
                        dst_all.at[pl.ds(rw, 1)])

    plsc.subcore_barrier()

    def fire_g(g):
        for b in range(K_PIPE):
            pltpu.async_copy(tbl.at[src_all.at[g * K_PIPE + b]],
                             rows_v.at[b], gsem)

    def drain_g(b):
        pltpu.make_async_copy(zeros_hbm.at[pl.ds(0, CHUNK)],
                              rows_v.at[b], gsem).wait()

    fire_g(0)

    def body(g, carry):
        sh = []
        for b in range(K_PIPE):
            drain_g(b)
            sh.append(pltpu.async_copy(
                rows_v.at[b], acc.at[dst_all.at[g * K_PIPE + b]],
                ssem, add=True))
        for b in range(K_PIPE):
            sh[b].wait()
            pltpu.async_copy(tbl.at[src_all.at[(g + 1) * K_PIPE + b]],
                             rows_v.at[b], gsem)
        return carry

    lax.fori_loop(0, groups - 1, body, 0)
    g_last = groups - 1
    sh = []
    for b in range(K_PIPE):
        drain_g(b)
        sh.append(pltpu.async_copy(
            rows_v.at[b], acc.at[dst_all.at[g_last * K_PIPE + b]],
            ssem, add=True))
    for h in sh:
        h.wait()

    @pl.when(wid < rem)
    def _():
        pltpu.async_copy(tbl.at[src_all.at[rw]], rows_v.at[0], gsem).wait()
        pltpu.sync_copy(rows_v.at[0], acc.at[dst_all.at[rw]], add=True)

    plsc.subcore_barrier()
    pltpu.sync_copy(acc.at[pl.ds(r0, RPS)],
                    out_hbm.at[pl.ds(cid * ACC_ROWS + r0, RPS)])


@functools.cache
def _sc_kernels(er, n_tbl):
    rw = er // NW
    assert rw % K_PIPE == 0
    mesh = plsc.VectorSubcoreMesh(
        core_axis_name="c", subcore_axis_name="s",
        num_cores=NC, num_subcores=NS)
    params = pltpu.CompilerParams(use_tc_tiling_on_sc=False)
    deg_kernel = pl.kernel(
        _deg_body,
        out_type=jax.ShapeDtypeStruct((NC * ACC_ROWS, H), jnp.float32),
        mesh=mesh,
        compiler_params=params,
        scratch_types=[
            pltpu.VMEM((rw + 1, CHUNK), jnp.int32),
            pltpu.VMEM((CHUNK, H), jnp.float32),
            pltpu.VMEM_SHARED((ACC_ROWS, H), jnp.float32),
            pltpu.SemaphoreType.DMA,
            pltpu.SemaphoreType.DMA,
        ],
    )
    segsum_kernel = pl.kernel(
        _segsum_body,
        out_type=jax.ShapeDtypeStruct((NC * ACC_ROWS, H), jnp.float32),
        mesh=mesh,
        compiler_params=params,
        scratch_types=[
            pltpu.VMEM((rw + 1, CHUNK), jnp.int32),
            pltpu.VMEM((rw + 1, CHUNK), jnp.int32),
            pltpu.VMEM((K_PIPE, CHUNK, H), jnp.float32),
            pltpu.VMEM_SHARED((ACC_ROWS, H), jnp.float32),
            pltpu.VMEM_SHARED((n_tbl, H), jnp.float32),
            pltpu.SemaphoreType.DMA,
            pltpu.SemaphoreType.DMA,
            pltpu.SemaphoreType.DMA,
        ],
    )
    return deg_kernel, segsum_kernel


BLKP = PR // 4


def _norm_phys(d0, d1):
    deg = d0 + d1
    return jnp.where(deg > 0.0, lax.rsqrt(deg), 0.0)


def _k1_body(d0_ref, d1_ref, x4_ref, w4_ref, a1_ref):
    norm = _norm_phys(d0_ref[...], d1_ref[...])
    a1_ref[...] = jnp.dot(x4_ref[...], w4_ref[...],
                          preferred_element_type=jnp.float32) * norm


def _k2_body(u0_ref, u1_ref, d0_ref, d1_ref, w4_ref, a2_ref):
    deg = d0_ref[...] + d1_ref[...]
    inv = jnp.where(deg > 0.0, 1.0 / deg, 0.0)
    u = u0_ref[...] + u1_ref[...]
    a2_ref[...] = jnp.dot(u, w4_ref[...],
                          preferred_element_type=jnp.float32) * inv


def _k3_body(u0_ref, u1_ref, d0_ref, d1_ref, n_ref, y_ref):
    norm = _norm_phys(d0_ref[...], d1_ref[...])
    g = jnp.maximum((u0_ref[...] + u1_ref[...]) * norm, 0.0)
    gs = jnp.concatenate([g[:, 16:], g[:, :16]], axis=1)
    lane = lax.broadcasted_iota(jnp.int32, g.shape, 1)
    y_ref[...] = jnp.where((lane % 32) < 16,
                           n_ref[...] * jnp.exp(gs) + g, 0.0)


BM = 512


def _k4_body(ytb_ref, yt_ref, out_ref):
    out_ref[...] = lax.dot_general(ytb_ref[...], yt_ref[...],
                                   (((0,), (0,)), ((), ())),
                                   preferred_element_type=jnp.float32)


def _dense_stage1(d0, d1, x4, w4):
    return pl.pallas_call(
        _k1_body,
        grid=(PR // BLKP,),
        in_specs=[
            pl.BlockSpec((BLKP, 128), lambda i: (i, 0)),
            pl.BlockSpec((BLKP, 128), lambda i: (i, 0)),
            pl.BlockSpec((BLKP, 4 * F_IN), lambda i: (i, 0)),
            pl.BlockSpec((4 * F_IN, 128), lambda i: (0, 0)),
        ],
        out_specs=pl.BlockSpec((BLKP, 128), lambda i: (i, 0)),
        out_shape=jax.ShapeDtypeStruct((PR, 128), jnp.float32),
    )(d0, d1, x4, w4)


def _dense_stage2(u0, u1, d0, d1, w4):
    return pl.pallas_call(
        _k2_body,
        grid=(PR // BLKP,),
        in_specs=[
            pl.BlockSpec((BLKP, 128), lambda i: (i, 0)),
            pl.BlockSpec((BLKP, 128), lambda i: (i, 0)),
            pl.BlockSpec((BLKP, 128), lambda i: (i, 0)),
            pl.BlockSpec((BLKP, 128), lambda i: (i, 0)),
            pl.BlockSpec((128, 128), lambda i: (0, 0)),
        ],
        out_specs=pl.BlockSpec((BLKP, 128), lambda i: (i, 0)),
        out_shape=jax.ShapeDtypeStruct((PR, 128), jnp.float32),
    )(u0, u1, d0, d1, w4)


def _dense_stage3(u0, u1, d0, d1, noise_p):
    return pl.pallas_call(
        _k3_body,
        grid=(PR // BLKP,),
        in_specs=[
            pl.BlockSpec((BLKP, 128), lambda i: (i, 0)),
            pl.BlockSpec((BLKP, 128), lambda i: (i, 0)),
            pl.BlockSpec((BLKP, 128), lambda i: (i, 0)),
            pl.BlockSpec((BLKP, 128), lambda i: (i, 0)),
            pl.BlockSpec((BLKP, 128), lambda i: (i, 0)),
        ],
        out_specs=pl.BlockSpec((BLKP, 128), lambda i: (i, 0)),
        out_shape=jax.ShapeDtypeStruct((PR, 128), jnp.float32),
    )(u0, u1, d0, d1, noise_p)


def _dense_stage4(yt):
    return pl.pallas_call(
        _k4_body,
        grid=(pl.cdiv(N, BM),),
        in_specs=[
            pl.BlockSpec((H, BM), lambda i: (0, i)),
            pl.BlockSpec((H, N), lambda i: (0, 0)),
        ],
        out_specs=pl.BlockSpec((BM, N), lambda i: (i, 0)),
        out_shape=jax.ShapeDtypeStruct((N, N), jnp.float32),
    )(yt, yt)


def kernel(features, edge_index, W_base, W_mean, W_logstd):
    e = edge_index.shape[1]
    assert e % CHUNK == 0
    edge3 = edge_index.reshape(2, e // CHUNK, CHUNK)

    zeros_h = jnp.zeros((ACC_ROWS, H), jnp.float32)
    ones_d = jnp.ones((CHUNK, H), jnp.float32)
    eye4 = jnp.eye(4, dtype=jnp.float32)
    w4b = jnp.kron(eye4, W_base)
    w4c = jnp.kron(eye4, jnp.concatenate([W_mean, W_logstd], axis=1))
    noise = jax.random.normal(jax.random.key(42), (N, DZ), jnp.float32)
    noise_p = jnp.zeros((ACC_ROWS, H), jnp.float32)
    noise_p = noise_p.at[:N, :DZ].set(noise).reshape(PR, 128)
    x4 = features.reshape(N // 4, 4 * F_IN)

    deg_kernel, segsum_kernel = _sc_kernels(e // CHUNK, ACC_ROWS)
    degp = deg_kernel(edge3, ones_d, zeros_h)
    d0 = degp[:ACC_ROWS].reshape(PR, 128)
    d1 = degp[ACC_ROWS:].reshape(PR, 128)
    d0, d1 = lax.optimization_barrier((d0, d1))
    x4p = jnp.zeros((PR, 4 * F_IN), jnp.float32).at[:N // 4].set(x4)
    a1 = _dense_stage1(d0, d1, x4p, w4b).reshape(ACC_ROWS, H)
    u1p = segsum_kernel(a1, edge3, zeros_h)
    a2 = _dense_stage2(u1p[:ACC_ROWS].reshape(PR, 128),
                       u1p[ACC_ROWS:].reshape(PR, 128),
                       d0, d1, w4c).reshape(ACC_ROWS, H)
    u2p = segsum_kernel(a2, edge3, zeros_h)
    y = _dense_stage3(u2p[:ACC_ROWS].reshape(PR, 128),
                      u2p[ACC_ROWS:].reshape(PR, 128),
                      d0, d1, noise_p)
    yt = y.reshape(ACC_ROWS, H)[:N].T
    return _dense_stage4(yt)
